# Initial kernel scaffold; baseline (speedup 1.0000x reference)
import jax
import jax.numpy as jnp
from jax.experimental import pallas as pl


def kernel(f0, f1, f2, a0, a1, a2, W1_0, b1_0, W2_0, b2_0, W1_1, b1_1, W2_1, b2_1, W1_2, b1_2, W2_2, b2_2):
    raise NotImplementedError("write your pallas kernel here")


# speedup vs baseline: 6.7033x; 6.7033x over previous
import math

import jax
import jax.numpy as jnp
from jax.experimental import pallas as pl
from jax.experimental.pallas import tpu as pltpu

_NUM_ANCHOR = 3
_NUM_PROPOSALS = 1000
_IOU_THRESH = 0.7
_C_IN = 256
_LOG_MAX = math.log(1000.0 / 16.0)
_P = 1024
_NTOT = 16128
_NPAD = 16384
_CHUNK = 2048
_HI = jax.lax.Precision.HIGHEST


def _conv_head_kernel(H, W):
    Wp = W + 2
    M = H * Wp
    F = (H + 4) * Wp

    def body(x_ref, w1_ref, b1_ref, w2_ref, b2_ref, o_ref, hid_ref):
        dn = (((1,), (0,)), ((), ()))
        y1 = jnp.zeros((M, _C_IN), jnp.float32)
        for t in range(9):
            dh, dw = divmod(t, 3)
            start = (dh + 1) * Wp + dw - 1
            y1 = y1 + jax.lax.dot_general(
                x_ref[0, pl.ds(start, M), :], w1_ref[t], dn,
                preferred_element_type=jnp.float32)
        y1 = jnp.maximum(y1 + b1_ref[...], 0.0)
        ci = jax.lax.rem(jax.lax.broadcasted_iota(jnp.int32, (M, 1), 0),
                         Wp)
        y1 = jnp.where((ci >= 1) & (ci <= W), y1, 0.0)
        hid_ref[...] = jnp.zeros((F, _C_IN), jnp.float32)
        hid_ref[pl.ds(2 * Wp, M), :] = y1
        y2 = jnp.zeros((M, 128), jnp.float32)
        for t in range(9):
            dh, dw = divmod(t, 3)
            start = (dh + 1) * Wp + dw - 1
            y2 = y2 + jax.lax.dot_general(
                hid_ref[pl.ds(start, M), :], w2_ref[t], dn,
                preferred_element_type=jnp.float32)
        o_ref[0] = y2 + b2_ref[...]

    return body, M, F


def _conv_level(x, W1, b1, W2, b2):
    B, C, H, W = x.shape
    Wp = W + 2
    body, M, F = _conv_head_kernel(H, W)
    x_flat = jnp.pad(jnp.transpose(x, (0, 2, 3, 1)),
                     ((0, 0), (2, 2), (1, 1), (0, 0))).reshape(B, F, C)
    w1t = jnp.transpose(W1, (2, 3, 1, 0)).reshape(9, C, C)
    w2t = jnp.transpose(W2, (2, 3, 1, 0)).reshape(9, C, 15)
    w2t = jnp.pad(w2t, ((0, 0), (0, 0), (0, 113)))
    b2p = jnp.pad(b2, (0, 113)).reshape(1, 128)
    out = pl.pallas_call(
        body,
        grid=(B,),
        in_specs=[
            pl.BlockSpec((1, F, C), lambda b: (b, 0, 0)),
            pl.BlockSpec((9, C, C), lambda b: (0, 0, 0)),
            pl.BlockSpec((1, C), lambda b: (0, 0)),
            pl.BlockSpec((9, C, 128), lambda b: (0, 0, 0)),
            pl.BlockSpec((1, 128), lambda b: (0, 0)),
        ],
        out_specs=pl.BlockSpec((1, M, 128), lambda b: (b, 0, 0)),
        out_shape=jax.ShapeDtypeStruct((B, M, 128), jnp.float32),
        scratch_shapes=[pltpu.VMEM((F, C), jnp.float32)],
    )(x_flat, w1t, b1.reshape(1, C), w2t, b2p)
    logits = out.reshape(B, H, Wp, 128)[:, :, 1:W + 1, :15]
    logits = logits.reshape(B, H * W * _NUM_ANCHOR, 5)
    return logits.at[..., 4].set(jax.nn.sigmoid(logits[..., 4]))


def _from_bytes(b, axis):
    n = b.shape[axis] // 4
    parts = jnp.split(b.astype(jnp.int32), 4, axis=axis)
    word = parts[0]
    for k in range(1, 4):
        word = jnp.bitwise_or(word, jax.lax.shift_left(parts[k], 8 * k))
    return jax.lax.bitcast_convert_type(word, jnp.float32)


def _to_bytes(x, axis):
    w = jax.lax.bitcast_convert_type(x, jnp.int32)
    parts = [jnp.bitwise_and(jax.lax.shift_right_logical(w, 8 * k), 255)
             for k in range(4)]
    return jnp.concatenate(parts, axis=axis).astype(jnp.float32)


def _select_nms_kernel(rows_t_ref, or_ref, o_ref):
    dn = (((1,), (0,)), ((), ()))
    orow = or_ref[0]
    selb_t = jnp.zeros((64, _P), jnp.float32)
    for c in range(_NPAD // _CHUNK):
        jj = jax.lax.broadcasted_iota(jnp.int32, (_CHUNK, 1), 0) + c * _CHUNK
        gt = (jj == orow).astype(jnp.float32)
        selb_t = selb_t + jax.lax.dot_general(
            rows_t_ref[0, :, pl.ds(c * _CHUNK, _CHUNK)], gt, dn,
            preferred_element_type=jnp.float32)
    sel_t = _from_bytes(selb_t, 0)
    sel = jnp.transpose(sel_t)

    def decode(d0, d1, d2, d3, ax1, ay1, ax2, ay2):
        wa = ax2 - ax1
        ha = ay2 - ay1
        cxa = ax1 + 0.5 * wa
        cya = ay1 + 0.5 * ha
        dx = d0 / 10.0
        dy = d1 / 10.0
        dw = jnp.minimum(d2 / 5.0, _LOG_MAX)
        dh = jnp.minimum(d3 / 5.0, _LOG_MAX)
        pcx = dx * wa + cxa
        pcy = dy * ha + cya
        pw = jnp.exp(dw) * wa
        ph = jnp.exp(dh) * ha
        return (pcx - 0.5 * pw, pcy - 0.5 * ph,
                pcx + 0.5 * pw, pcy + 0.5 * ph)

    x1c, y1c, x2c, y2c = decode(*(sel[:, k:k + 1] for k in range(4)),
                                *(sel[:, k:k + 1] for k in range(6, 10)))
    sc_c = sel[:, 4:5]
    aid_c = sel[:, 5:6]
    x1r, y1r, x2r, y2r = decode(*(sel_t[k:k + 1, :] for k in range(4)),
                                *(sel_t[k:k + 1, :] for k in range(6, 10)))

    area_c = (x2c - x1c) * (y2c - y1c)
    area_r = (x2r - x1r) * (y2r - y1r)
    iw = jnp.maximum(jnp.minimum(x2c, x2r) - jnp.maximum(x1c, x1r), 0.0)
    ih = jnp.maximum(jnp.minimum(y2c, y2r) - jnp.maximum(y1c, y1r), 0.0)
    inter = iw * ih
    iou = inter / (area_c + area_r - inter + 1e-9)

    ic = jax.lax.broadcasted_iota(jnp.int32, (_P, _P), 0)
    jr = jax.lax.broadcasted_iota(jnp.int32, (_P, _P), 1)
    a_mat = ((iou > _IOU_THRESH) & (ic < jr)).astype(jnp.float32)
    valid = (jax.lax.broadcasted_iota(jnp.int32, (1, _P), 1)
             < _NUM_PROPOSALS).astype(jnp.float32)

    def cond(carry):
        return carry[1]

    def step(carry):
        k, _ = carry
        sup = jax.lax.dot_general(k, a_mat, dn,
                                  preferred_element_type=jnp.float32)
        k2 = jnp.where(sup > 0.0, 0.0, valid)
        return k2, jnp.any(k2 != k)

    keep, _ = jax.lax.while_loop(cond, step, (valid, True))

    lt = (ic <= jr).astype(jnp.float32)
    pos = jax.lax.dot_general(keep, lt, dn,
                              preferred_element_type=jnp.float32)
    pos_c = jnp.transpose(pos)
    keep_c = jnp.transpose(keep)
    pf = jr.astype(jnp.float32)
    perm_t = ((pos_c - 1.0 == pf) & (keep_c > 0.5)).astype(jnp.float32)
    z = jnp.zeros((1, _P), jnp.float32)
    sel6_t = jnp.concatenate(
        [x1r, y1r, x2r, y2r, sel_t[4:5, :], sel_t[5:6, :], z, z], axis=0)
    out_bt = jax.lax.dot_general(_to_bytes(sel6_t, 0), perm_t, dn,
                                 preferred_element_type=jnp.float32)
    o_ref[0] = jnp.transpose(_from_bytes(out_bt, 0))


def kernel(f0, f1, f2, a0, a1, a2, W1_0, b1_0, W2_0, b2_0,
           W1_1, b1_1, W2_1, b2_1, W1_2, b1_2, W2_2, b2_2):
    B = f0.shape[0]
    logits = [
        _conv_level(f0, W1_0, b1_0, W2_0, b2_0),
        _conv_level(f1, W1_1, b1_1, W2_1, b2_1),
        _conv_level(f2, W1_2, b1_2, W2_2, b2_2),
    ]
    aid = jnp.concatenate([
        jnp.tile(jnp.arange(3 * i, 3 * i + 3, dtype=jnp.float32),
                 (lg.shape[1] // _NUM_ANCHOR,))
        for i, lg in enumerate(logits)
    ])
    lg = jnp.concatenate(logits, axis=1)
    anchors = jnp.concatenate([a0, a1, a2], axis=0)
    rows = jnp.concatenate([
        lg,
        jnp.broadcast_to(aid[None, :, None], (B, _NTOT, 1)),
        jnp.broadcast_to(anchors[None], (B, _NTOT, 4)),
    ], axis=-1)
    rows = jnp.pad(rows, ((0, 0), (0, _NPAD - _NTOT), (0, 6)))
    w = jax.lax.bitcast_convert_type(rows, jnp.int32)
    rows_bt = jnp.transpose(jnp.concatenate(
        [jnp.bitwise_and(jax.lax.shift_right_logical(w, 8 * k), 255)
         for k in range(4)], axis=-1).astype(jnp.float32),
        (0, 2, 1))

    _, order = jax.lax.top_k(lg[..., 4], _NUM_PROPOSALS)
    order = jnp.pad(order, ((0, 0), (0, _P - _NUM_PROPOSALS)))

    out = pl.pallas_call(
        _select_nms_kernel,
        grid=(B,),
        in_specs=[
            pl.BlockSpec((1, 64, _NPAD), lambda b: (b, 0, 0)),
            pl.BlockSpec((1, 1, _P), lambda b: (b, 0, 0)),
        ],
        out_specs=pl.BlockSpec((1, _P, 8), lambda b: (b, 0, 0)),
        out_shape=jax.ShapeDtypeStruct((B, _P, 8), jnp.float32),
    )(rows_bt, order[:, None, :])
    return out[:, :_NUM_PROPOSALS, :6]
